# Initial kernel scaffold; baseline (speedup 1.0000x reference)
import jax
import jax.numpy as jnp
from jax.experimental import pallas as pl


def kernel(x, coefficient, bias):
    raise NotImplementedError("write your pallas kernel here")


# speedup vs baseline: 1.3509x; 1.3509x over previous
import jax
import jax.numpy as jnp
from jax.experimental import pallas as pl
from jax.experimental.pallas import tpu as pltpu

_C = 8


def _pool_body(s_ref, x_ref, p_ref, o_ref):
    v = x_ref[...]
    c, hh, w2 = v.shape
    w = w2 // 2
    hp = v[:, :, :w] + v[:, :, w:]
    hp2 = hp.reshape(c * hh, w)
    wp = jnp.dot(hp2, p_ref[...], preferred_element_type=jnp.float32)
    o_ref[...] = (s_ref[0] * wp + s_ref[1]).reshape(c, hh, w // 2)


def kernel(x, coefficient, bias):
    b, c, h, w = x.shape
    oh, ow = h // 2, w // 2
    xf = x.reshape(b * c, h // 2, 2 * w)
    scale = jnp.concatenate([coefficient, bias])
    pair = jnp.repeat(jnp.eye(w // 2, dtype=x.dtype), 2, axis=0)
    out = pl.pallas_call(
        _pool_body,
        grid=(b * c // _C,),
        in_specs=[
            pl.BlockSpec(memory_space=pltpu.SMEM),
            pl.BlockSpec((_C, h // 2, 2 * w), lambda i: (i, 0, 0)),
            pl.BlockSpec((w, w // 2), lambda i: (0, 0)),
        ],
        out_specs=pl.BlockSpec((_C, oh, ow), lambda i: (i, 0, 0)),
        out_shape=jax.ShapeDtypeStruct((b * c, oh, ow), x.dtype),
        compiler_params=pltpu.CompilerParams(
            dimension_semantics=("parallel",),
            vmem_limit_bytes=48 * 1024 * 1024,
        ),
    )(scale, xf, pair)
    return out.reshape(b, c, oh, ow)


# speedup vs baseline: 4.3380x; 3.2113x over previous
import jax
import jax.numpy as jnp
from jax.experimental import pallas as pl
from jax.experimental.pallas import tpu as pltpu

_C = 8


def _pool_body(s_ref, x_ref, p_ref, o_ref):
    v = x_ref[...]
    c, h, w = v.shape
    wp = jnp.dot(v.reshape(c * h, w), p_ref[...],
                 preferred_element_type=jnp.float32)
    wr = wp.reshape(c, h // 2, 2, w // 2)
    hp = wr[:, :, 0, :] + wr[:, :, 1, :]
    o_ref[...] = s_ref[0] * hp + s_ref[1]


def kernel(x, coefficient, bias):
    b, c, h, w = x.shape
    oh, ow = h // 2, w // 2
    xf = x.reshape(b * c, h, w)
    scale = jnp.concatenate([coefficient, bias])
    pair = jnp.repeat(jnp.eye(w // 2, dtype=x.dtype), 2, axis=0)
    out = pl.pallas_call(
        _pool_body,
        grid=(b * c // _C,),
        in_specs=[
            pl.BlockSpec(memory_space=pltpu.SMEM),
            pl.BlockSpec((_C, h, w), lambda i: (i, 0, 0)),
            pl.BlockSpec((w, w // 2), lambda i: (0, 0)),
        ],
        out_specs=pl.BlockSpec((_C, oh, ow), lambda i: (i, 0, 0)),
        out_shape=jax.ShapeDtypeStruct((b * c, oh, ow), x.dtype),
        compiler_params=pltpu.CompilerParams(
            dimension_semantics=("parallel",),
            vmem_limit_bytes=48 * 1024 * 1024,
        ),
    )(scale, xf, pair)
    return out.reshape(b, c, oh, ow)


# speedup vs baseline: 4.7929x; 1.1049x over previous
import jax
import jax.numpy as jnp
from jax.experimental import pallas as pl
from jax.experimental.pallas import tpu as pltpu

_C = 16


def _pool_body(s_ref, x_ref, p_ref, o_ref):
    v = x_ref[...]
    c, h, w = v.shape
    wp = jnp.dot(v.reshape(c * h, w), p_ref[...],
                 preferred_element_type=jnp.float32)
    wr = wp.reshape(c, h // 2, 2, w // 2)
    hp = wr[:, :, 0, :] + wr[:, :, 1, :]
    o_ref[...] = s_ref[0] * hp + s_ref[1]


def kernel(x, coefficient, bias):
    b, c, h, w = x.shape
    oh, ow = h // 2, w // 2
    xf = x.reshape(b * c, h, w)
    scale = jnp.concatenate([coefficient, bias])
    pair = jnp.repeat(jnp.eye(w // 2, dtype=x.dtype), 2, axis=0)
    out = pl.pallas_call(
        _pool_body,
        grid=(b * c // _C,),
        in_specs=[
            pl.BlockSpec(memory_space=pltpu.SMEM),
            pl.BlockSpec((_C, h, w), lambda i: (i, 0, 0)),
            pl.BlockSpec((w, w // 2), lambda i: (0, 0)),
        ],
        out_specs=pl.BlockSpec((_C, oh, ow), lambda i: (i, 0, 0)),
        out_shape=jax.ShapeDtypeStruct((b * c, oh, ow), x.dtype),
        compiler_params=pltpu.CompilerParams(
            dimension_semantics=("parallel",),
            vmem_limit_bytes=48 * 1024 * 1024,
        ),
    )(scale, xf, pair)
    return out.reshape(b, c, oh, ow)
